# baseline (speedup 1.0000x reference)
import functools
import math

import jax
import jax.numpy as jnp
from jax import lax
from jax.experimental import pallas as pl
from jax.experimental.pallas import tpu as pltpu
from jax.experimental.pallas import tpu_sc as plsc

_NC = 2
_NS = 16
_LANES = 16
_NW = _NC * _NS

_CHUNK = 1024


@functools.lru_cache(maxsize=None)
def _build(n, d, scale):
    assert n % (_NW * _CHUNK) == 0 and d % _LANES == 0
    per_w = n // _NW
    n_chunks = per_w // _CHUNK
    mesh = plsc.VectorSubcoreMesh(core_axis_name="c", subcore_axis_name="s")

    @functools.partial(
        pl.kernel,
        out_type=jax.ShapeDtypeStruct((n, d), jnp.float32),
        mesh=mesh,
        scratch_types=[
            pltpu.VMEM((_CHUNK,), jnp.int32),
            pltpu.VMEM((_CHUNK, d), jnp.float32),
            pltpu.SemaphoreType.DMA,
        ],
        compiler_params=pltpu.CompilerParams(use_tc_tiling_on_sc=False),
    )
    def emb(table_hbm, idx_hbm, out_hbm, idx_v, rows_v, sem):
        wid = lax.axis_index("s") * _NC + lax.axis_index("c")
        base = wid * per_w

        @pl.loop(0, n_chunks)
        def _(g):
            off = base + g * _CHUNK
            pltpu.sync_copy(idx_hbm.at[pl.ds(off, _CHUNK)], idx_v)
            pltpu.async_copy(table_hbm.at[idx_v], rows_v, sem).wait()

            @pl.loop(0, _CHUNK)
            def _(i):
                for j in range(d // _LANES):
                    sl = pl.ds(j * _LANES, _LANES)
                    rows_v[i, sl] = rows_v[i, sl] * scale

            pltpu.sync_copy(rows_v, out_hbm.at[pl.ds(off, _CHUNK)])

    return emb


def kernel(x, table):
    b, l = x.shape
    _, d = table.shape
    idx = x.reshape(-1).astype(jnp.int32)
    scale = math.sqrt(d)
    out = _build(b * l, d, scale)(table, idx)
    return out.reshape(b, l, d)


# speedup vs baseline: 1.1050x; 1.1050x over previous
import functools
import math

import jax
import jax.numpy as jnp
from jax import lax
from jax.experimental import pallas as pl
from jax.experimental.pallas import tpu as pltpu
from jax.experimental.pallas import tpu_sc as plsc

_NC = 2
_NS = 16
_LANES = 16
_NW = _NC * _NS

_CHUNK = 800
_NBUF = 2


@functools.lru_cache(maxsize=None)
def _build(n, d, scale):
    assert n % (_NW * _CHUNK * _NBUF) == 0 and d % _LANES == 0
    per_w = n // _NW
    n_chunks = per_w // _CHUNK
    mesh = plsc.VectorSubcoreMesh(core_axis_name="c", subcore_axis_name="s")

    @functools.partial(
        pl.kernel,
        out_type=jax.ShapeDtypeStruct((n, d), jnp.float32),
        mesh=mesh,
        scratch_types=[
            pltpu.VMEM((per_w,), jnp.int32),
            [pltpu.VMEM((_CHUNK, d), jnp.float32) for _ in range(_NBUF)],
            [pltpu.SemaphoreType.DMA for _ in range(_NBUF)],
            [pltpu.SemaphoreType.DMA for _ in range(_NBUF)],
        ],
        compiler_params=pltpu.CompilerParams(use_tc_tiling_on_sc=False),
    )
    def emb(table_hbm, idx_hbm, out_hbm, idx_v, rows, gsem, osem):
        wid = lax.axis_index("s") * _NC + lax.axis_index("c")
        base = wid * per_w

        pltpu.sync_copy(idx_hbm.at[pl.ds(base, per_w)], idx_v)

        def start_gather(step, b):
            pltpu.async_copy(
                table_hbm.at[idx_v.at[pl.ds(step * _CHUNK, _CHUNK)]],
                rows[b],
                gsem[b],
            )

        for b in range(_NBUF):
            start_gather(b, b)

        @pl.loop(0, n_chunks, step=_NBUF)
        def _(g):
            for b in range(_NBUF):
                step = g + b
                pltpu.make_async_copy(
                    table_hbm.at[pl.ds(0, _CHUNK)], rows[b], gsem[b]
                ).wait()

                @pl.loop(0, _CHUNK, unroll=8)
                def _(i):
                    for j in range(d // _LANES):
                        sl = pl.ds(j * _LANES, _LANES)
                        rows[b][i, sl] = rows[b][i, sl] * scale

                out_slice = out_hbm.at[pl.ds(base + step * _CHUNK, _CHUNK)]
                pltpu.async_copy(rows[b], out_slice, osem[b])

                @pl.when(step + _NBUF < n_chunks)
                def _():
                    pltpu.make_async_copy(rows[b], out_slice, osem[b]).wait()
                    start_gather(step + _NBUF, b)

        for b in range(_NBUF):
            last = out_hbm.at[pl.ds(base, _CHUNK)]
            pltpu.make_async_copy(rows[b], last, osem[b]).wait()

    return emb


def kernel(x, table):
    b, l = x.shape
    _, d = table.shape
    idx = x.reshape(-1).astype(jnp.int32)
    out = _build(b * l, d, math.sqrt(d))(table, idx)
    return out.reshape(b, l, d)
